# baseline (speedup 1.0000x reference)
import numpy as np
import jax
import jax.numpy as jnp
from jax.experimental import pallas as pl
from jax.experimental.pallas import tpu as pltpu

R = 64
C = 100000
C_BLK = 8192
NBLK = (C + C_BLK - 1) // C_BLK

_U32 = jnp.uint32
_KEY_HI = np.uint32(0)
_KEY_LO = np.uint32(42)
_KS2 = np.uint32(np.uint32(0) ^ np.uint32(42) ^ np.uint32(0x1BD11BDA))
_TINY = np.float32(np.finfo(np.float32).tiny)


def _rotl(v, d):
    return (v << _U32(d)) | jax.lax.shift_right_logical(v, _U32(32 - d))


def _threefry_bits(idx):
    ks = (_KEY_HI, _KEY_LO, _KS2)
    x0 = jnp.full_like(idx, ks[0])
    x1 = idx + ks[1]

    def rounds(x0, x1, rots):
        for r in rots:
            x0 = x0 + x1
            x1 = _rotl(x1, r) ^ x0
        return x0, x1

    ra = (13, 15, 26, 6)
    rb = (17, 29, 16, 24)
    x0, x1 = rounds(x0, x1, ra)
    x0 = x0 + ks[1]
    x1 = x1 + ks[2] + _U32(1)
    x0, x1 = rounds(x0, x1, rb)
    x0 = x0 + ks[2]
    x1 = x1 + ks[0] + _U32(2)
    x0, x1 = rounds(x0, x1, ra)
    x0 = x0 + ks[0]
    x1 = x1 + ks[1] + _U32(3)
    x0, x1 = rounds(x0, x1, rb)
    x0 = x0 + ks[1]
    x1 = x1 + ks[2] + _U32(4)
    x0, x1 = rounds(x0, x1, ra)
    x0 = x0 + ks[2]
    x1 = x1 + ks[0] + _U32(5)
    return x0 ^ x1


def _sample_kernel(x_ref, o_ref, acc_val, acc_idx):
    j = pl.program_id(0)

    @pl.when(j == 0)
    def _():
        acc_val[...] = jnp.full_like(acc_val, -jnp.inf)
        acc_idx[...] = jnp.zeros_like(acc_idx)

    logits = x_ref[...]
    rows = jax.lax.broadcasted_iota(_U32, (R, C_BLK), 0)
    col = jax.lax.broadcasted_iota(_U32, (R, C_BLK), 1) + _U32(C_BLK) * j.astype(_U32)
    idx = rows * _U32(C) + col

    bits = _threefry_bits(idx)
    float_bits = jax.lax.shift_right_logical(bits, _U32(9)) | _U32(0x3F800000)
    f = jax.lax.bitcast_convert_type(float_bits, jnp.float32) - np.float32(1.0)
    u = jnp.maximum(_TINY, f * (np.float32(1.0) - _TINY) + _TINY)
    g = -jnp.log(-jnp.log(u))

    score = g + logits
    score = jnp.where(col < _U32(C), score, -jnp.inf)

    blk_max = jnp.max(score, axis=1, keepdims=True)
    cand = jnp.where(score == blk_max, col.astype(jnp.int32), jnp.int32(np.iinfo(np.int32).max))
    blk_arg = jnp.min(cand, axis=1, keepdims=True)

    better = blk_max > acc_val[...]
    acc_idx[...] = jnp.where(better, blk_arg, acc_idx[...])
    acc_val[...] = jnp.where(better, blk_max, acc_val[...])

    @pl.when(j == NBLK - 1)
    def _():
        o_ref[...] = acc_idx[...]


def kernel(logits):
    out = pl.pallas_call(
        _sample_kernel,
        grid=(NBLK,),
        in_specs=[pl.BlockSpec((R, C_BLK), lambda j: (0, j))],
        out_specs=pl.BlockSpec((R, 1), lambda j: (0, 0)),
        out_shape=jax.ShapeDtypeStruct((R, 1), jnp.int32),
        scratch_shapes=[
            pltpu.VMEM((R, 1), jnp.float32),
            pltpu.VMEM((R, 1), jnp.int32),
        ],
        compiler_params=pltpu.CompilerParams(
            dimension_semantics=("arbitrary",),
        ),
    )(logits)
    return out.reshape(R)


# speedup vs baseline: 1.3934x; 1.3934x over previous
import numpy as np
import jax
import jax.numpy as jnp
from jax.experimental import pallas as pl
from jax.experimental.pallas import tpu as pltpu

R = 64
C = 100000
C_BLK = 8192
NBLK = (C + C_BLK - 1) // C_BLK
W = 1024
CHUNKS = C_BLK // W
RG = R // 8

_U32 = jnp.uint32
_KS = (np.uint32(0), np.uint32(42), np.uint32(42 ^ 0x1BD11BDA))
_TINY = np.float32(np.finfo(np.float32).tiny)
_IMAX = np.int32(np.iinfo(np.int32).max)


def _rotl(v, d):
    return (v << _U32(d)) | jax.lax.shift_right_logical(v, _U32(32 - d))


def _threefry_bits(idx_u32):
    a = idx_u32 + _KS[1]
    x0 = a
    x1 = _rotl(a, 13) ^ a

    def rounds(x0, x1, rots):
        for r in rots:
            x0 = x0 + x1
            x1 = _rotl(x1, r) ^ x0
        return x0, x1

    x0, x1 = rounds(x0, x1, (15, 26, 6))
    x0 = x0 + _KS[1]
    x1 = x1 + np.uint32(_KS[2] + np.uint32(1))
    x0, x1 = rounds(x0, x1, (17, 29, 16, 24))
    x0 = x0 + _KS[2]
    x1 = x1 + np.uint32(_KS[0] + np.uint32(2))
    x0, x1 = rounds(x0, x1, (13, 15, 26, 6))
    x0 = x0 + _KS[0]
    x1 = x1 + np.uint32(_KS[1] + np.uint32(3))
    x0, x1 = rounds(x0, x1, (17, 29, 16, 24))
    x0 = x0 + _KS[1]
    x1 = x1 + np.uint32(_KS[2] + np.uint32(4))
    x0, x1 = rounds(x0, x1, (13, 15, 26, 6))
    x0 = x0 + _KS[2]
    x1 = x1 + np.uint32(_KS[0] + np.uint32(5))
    return x0 ^ x1


def _sample_kernel(x_ref, o_ref, acc_val, acc_idx):
    j = pl.program_id(0)

    @pl.when(j == 0)
    def _():
        acc_val[...] = jnp.full((R, W), -jnp.inf, jnp.float32)
        acc_idx[...] = jnp.full((R, W), _IMAX, jnp.int32)

    col_local = jax.lax.broadcasted_iota(jnp.int32, (8, W), 1)
    row_base = jax.lax.broadcasted_iota(jnp.int32, (8, W), 0) * np.int32(C)
    blk0 = j * np.int32(C_BLK)

    for rg in range(RG):
        av = acc_val[pl.ds(rg * 8, 8), :]
        ac = acc_idx[pl.ds(rg * 8, 8), :]
        row_c = row_base + np.int32(rg * 8 * C)
        for ck in range(CHUNKS):
            logits = x_ref[pl.ds(rg * 8, 8), pl.ds(ck * W, W)]
            colg = col_local + (blk0 + np.int32(ck * W))
            idx = colg + row_c
            bits = _threefry_bits(jax.lax.bitcast_convert_type(idx, _U32))
            m = jax.lax.shift_right_logical(bits, _U32(9))
            f = jax.lax.convert_element_type(m, jnp.float32) * np.float32(2.0 ** -23)
            u = jnp.maximum(_TINY, f)
            nl2 = jnp.log(-jnp.log(u))
            score = logits - nl2
            take = (score > av) & (colg < np.int32(C))
            av = jnp.where(take, score, av)
            ac = jnp.where(take, colg, ac)
        acc_val[pl.ds(rg * 8, 8), :] = av
        acc_idx[pl.ds(rg * 8, 8), :] = ac

        @pl.when(j == NBLK - 1)
        def _():
            rmax = jnp.max(av, axis=1, keepdims=True)
            cand = jnp.where(av == rmax, ac, _IMAX)
            o_ref[pl.ds(rg * 8, 8), :] = jnp.min(cand, axis=1, keepdims=True)


def kernel(logits):
    out = pl.pallas_call(
        _sample_kernel,
        grid=(NBLK,),
        in_specs=[pl.BlockSpec((R, C_BLK), lambda j: (0, j))],
        out_specs=pl.BlockSpec((R, 1), lambda j: (0, 0)),
        out_shape=jax.ShapeDtypeStruct((R, 1), jnp.int32),
        scratch_shapes=[
            pltpu.VMEM((R, W), jnp.float32),
            pltpu.VMEM((R, W), jnp.int32),
        ],
        compiler_params=pltpu.CompilerParams(
            dimension_semantics=("arbitrary",),
        ),
    )(logits)
    return out.reshape(R)


# speedup vs baseline: 1.4363x; 1.0308x over previous
import numpy as np
import jax
import jax.numpy as jnp
from jax.experimental import pallas as pl
from jax.experimental.pallas import tpu as pltpu

R = 64
C = 100000
C_BLK = 8192
NBLK = (C + C_BLK - 1) // C_BLK
W = 1024
CHUNKS = C_BLK // W
RG = R // 8

_U32 = jnp.uint32
_KS = (np.uint32(0), np.uint32(42), np.uint32(42 ^ 0x1BD11BDA))
_TINY = np.float32(np.finfo(np.float32).tiny)
_IMAX = np.int32(np.iinfo(np.int32).max)


def _rotl(v, d):
    return (v << _U32(d)) | jax.lax.shift_right_logical(v, _U32(32 - d))


def _threefry_bits(a):
    x0 = a
    x1 = _rotl(a, 13) ^ a

    def rounds(x0, x1, rots):
        for r in rots:
            x0 = x0 + x1
            x1 = _rotl(x1, r) ^ x0
        return x0, x1

    x0, x1 = rounds(x0, x1, (15, 26, 6))
    x0 = x0 + _KS[1]
    x1 = x1 + np.uint32(_KS[2] + np.uint32(1))
    x0, x1 = rounds(x0, x1, (17, 29, 16, 24))
    x0 = x0 + _KS[2]
    x1 = x1 + np.uint32(2)
    x0, x1 = rounds(x0, x1, (13, 15, 26, 6))
    x1 = x1 + np.uint32(_KS[1] + np.uint32(3))
    x0, x1 = rounds(x0, x1, (17, 29, 16, 24))
    x0 = x0 + _KS[1]
    x1 = x1 + np.uint32(_KS[2] + np.uint32(4))
    x0, x1 = rounds(x0, x1, (13, 15, 26, 6))
    x0 = x0 + _KS[2]
    x1 = x1 + np.uint32(5)
    return x0 ^ x1


def _sample_kernel(x_ref, o_ref, acc_val, acc_t):
    j = pl.program_id(0)

    @pl.when(j == 0)
    def _():
        acc_val[...] = jnp.full((R, W), -jnp.inf, jnp.float32)
        acc_t[...] = jnp.zeros((R, W), jnp.int32)

    lane = jax.lax.broadcasted_iota(jnp.int32, (8, W), 1)
    row_c = jax.lax.broadcasted_iota(jnp.int32, (8, W), 0) * np.int32(C)
    base42 = jax.lax.bitcast_convert_type(lane + row_c, _U32) + _U32(42)

    for rg in range(RG):
        av = acc_val[pl.ds(rg * 8, 8), :]
        at = acc_t[pl.ds(rg * 8, 8), :]
        rg_off = _U32(np.uint32(rg * 8 * C))
        for ck in range(CHUNKS):
            t = j * np.int32(CHUNKS) + np.int32(ck)
            logits = x_ref[pl.ds(rg * 8, 8), pl.ds(ck * W, W)]
            a = base42 + (t.astype(_U32) * _U32(W) + rg_off)
            bits = _threefry_bits(a)
            m = jax.lax.shift_right_logical(bits, _U32(9))
            f = jax.lax.convert_element_type(m, jnp.float32) * np.float32(2.0 ** -23)
            u = jnp.maximum(_TINY, f)
            nl2 = jnp.log(-jnp.log(u))
            score = logits - nl2
            take = (score > av) & (lane < C - t * np.int32(W))
            av = jnp.where(take, score, av)
            at = jnp.where(take, t, at)
        acc_val[pl.ds(rg * 8, 8), :] = av
        acc_t[pl.ds(rg * 8, 8), :] = at

    @pl.when(j == NBLK - 1)
    def _():
        for rg in range(RG):
            av = acc_val[pl.ds(rg * 8, 8), :]
            col = acc_t[pl.ds(rg * 8, 8), :] * np.int32(W) + lane
            rmax = jnp.max(av, axis=1, keepdims=True)
            cand = jnp.where(av == rmax, col, _IMAX)
            o_ref[pl.ds(rg * 8, 8), :] = jnp.min(cand, axis=1, keepdims=True)


def kernel(logits):
    out = pl.pallas_call(
        _sample_kernel,
        grid=(NBLK,),
        in_specs=[pl.BlockSpec((R, C_BLK), lambda j: (0, j))],
        out_specs=pl.BlockSpec((R, 1), lambda j: (0, 0)),
        out_shape=jax.ShapeDtypeStruct((R, 1), jnp.int32),
        scratch_shapes=[
            pltpu.VMEM((R, W), jnp.float32),
            pltpu.VMEM((R, W), jnp.int32),
        ],
        compiler_params=pltpu.CompilerParams(
            dimension_semantics=("arbitrary",),
        ),
    )(logits)
    return out.reshape(R)


# speedup vs baseline: 1.5492x; 1.0786x over previous
import numpy as np
import jax
import jax.numpy as jnp
from jax.experimental import pallas as pl
from jax.experimental.pallas import tpu as pltpu

R = 64
C = 100000
C_BLK = 16384
NBLK = (C + C_BLK - 1) // C_BLK
W = 1024
CHUNKS = C_BLK // W
RG = R // 8
TAIL = C - (NBLK - 1) * C_BLK
TAIL_FULL = TAIL // W
TAIL_REM = TAIL - TAIL_FULL * W

_U32 = jnp.uint32
_KS = (np.uint32(0), np.uint32(42), np.uint32(42 ^ 0x1BD11BDA))
_TINY = np.float32(np.finfo(np.float32).tiny)
_IMAX = np.int32(np.iinfo(np.int32).max)


def _rotl(v, d):
    return (v << _U32(d)) | jax.lax.shift_right_logical(v, _U32(32 - d))


def _threefry_bits(a):
    x0 = a
    x1 = _rotl(a, 13) ^ a

    def rounds(x0, x1, rots):
        for r in rots:
            x0 = x0 + x1
            x1 = _rotl(x1, r) ^ x0
        return x0, x1

    x0, x1 = rounds(x0, x1, (15, 26, 6))
    x0 = x0 + _KS[1]
    x1 = x1 + np.uint32(_KS[2] + np.uint32(1))
    x0, x1 = rounds(x0, x1, (17, 29, 16, 24))
    x0 = x0 + _KS[2]
    x1 = x1 + np.uint32(2)
    x0, x1 = rounds(x0, x1, (13, 15, 26, 6))
    x1 = x1 + np.uint32(_KS[1] + np.uint32(3))
    x0, x1 = rounds(x0, x1, (17, 29, 16, 24))
    x0 = x0 + _KS[1]
    x1 = x1 + np.uint32(_KS[2] + np.uint32(4))
    x0, x1 = rounds(x0, x1, (13, 15, 26, 6))
    x0 = x0 + _KS[2]
    x1 = x1 + np.uint32(5)
    return x0 ^ x1


def _score(x_ref, base42, rg, ck, t):
    logits = x_ref[pl.ds(rg * 8, 8), pl.ds(ck * W, W)]
    a = base42 + (t.astype(_U32) * _U32(W) + _U32(np.uint32(rg * 8 * C)))
    bits = _threefry_bits(a)
    m = jax.lax.shift_right_logical(bits, _U32(9))
    f = jax.lax.convert_element_type(m, jnp.float32) * np.float32(2.0 ** -23)
    u = jnp.maximum(_TINY, f)
    nl2 = jnp.log(-jnp.log(u))
    return logits - nl2


def _sample_kernel(x_ref, o_ref, acc_val, acc_t):
    j = pl.program_id(0)

    @pl.when(j == 0)
    def _():
        acc_val[...] = jnp.full((R, W), -jnp.inf, jnp.float32)
        acc_t[...] = jnp.zeros((R, W), jnp.int32)

    lane = jax.lax.broadcasted_iota(jnp.int32, (8, W), 1)
    row_c = jax.lax.broadcasted_iota(jnp.int32, (8, W), 0) * np.int32(C)
    base42 = jax.lax.bitcast_convert_type(lane + row_c, _U32) + _U32(42)

    @pl.when(j != NBLK - 1)
    def _():
        for rg in range(RG):
            av = acc_val[pl.ds(rg * 8, 8), :]
            at = acc_t[pl.ds(rg * 8, 8), :]
            for ck in range(CHUNKS):
                t = j * np.int32(CHUNKS) + np.int32(ck)
                score = _score(x_ref, base42, rg, ck, t)
                take = score > av
                av = jnp.where(take, score, av)
                at = jnp.where(take, t, at)
            acc_val[pl.ds(rg * 8, 8), :] = av
            acc_t[pl.ds(rg * 8, 8), :] = at

    @pl.when(j == NBLK - 1)
    def _():
        for rg in range(RG):
            av = acc_val[pl.ds(rg * 8, 8), :]
            at = acc_t[pl.ds(rg * 8, 8), :]
            for ck in range(TAIL_FULL + (1 if TAIL_REM else 0)):
                t = np.int32((NBLK - 1) * CHUNKS + ck)
                score = _score(x_ref, base42, rg, ck, jnp.int32(t))
                take = score > av
                if ck >= TAIL_FULL:
                    take = take & (lane < np.int32(TAIL_REM))
                av = jnp.where(take, score, av)
                at = jnp.where(take, t, at)
            col = at * np.int32(W) + lane
            rmax = jnp.max(av, axis=1, keepdims=True)
            cand = jnp.where(av == rmax, col, _IMAX)
            o_ref[pl.ds(rg * 8, 8), :] = jnp.min(cand, axis=1, keepdims=True)


def kernel(logits):
    out = pl.pallas_call(
        _sample_kernel,
        grid=(NBLK,),
        in_specs=[pl.BlockSpec((R, C_BLK), lambda j: (0, j))],
        out_specs=pl.BlockSpec((R, 1), lambda j: (0, 0)),
        out_shape=jax.ShapeDtypeStruct((R, 1), jnp.int32),
        scratch_shapes=[
            pltpu.VMEM((R, W), jnp.float32),
            pltpu.VMEM((R, W), jnp.int32),
        ],
        compiler_params=pltpu.CompilerParams(
            dimension_semantics=("arbitrary",),
        ),
    )(logits)
    return out.reshape(R)
